# baseline (speedup 1.0000x reference)
import functools

import jax
import jax.numpy as jnp
from jax import lax
from jax.experimental import pallas as pl
from jax.experimental.pallas import tpu as pltpu
from jax.experimental.pallas import tpu_sc as plsc

N = 50000
C = 10000
D = 768
LANES = 16
BLK = 40
NUM_BLOCKS = N // BLK
NCHUNK = D // LANES
NGRP = (BLK + LANES - 1) // LANES
EPS_PLAIN_SQ = 0.4 * 0.4
EPS_MODERATE_SQ = 0.6 * 0.6

_info = plsc.get_sparse_core_info()
NC = _info.num_cores
NS = _info.num_subcores
NW = NC * NS
BASE_NB = NUM_BLOCKS // NW
EXTRA = NUM_BLOCKS % NW
MAX_NB = BASE_NB + 1
BASE_ROWS = BASE_NB * BLK
MAX_ROWS = MAX_NB * BLK


def _tbgm_body(feats_hbm, mem_hbm, pid_hbm, out_hbm,
               idx_all, out_all, fb_a, fb_b, rb_a, rb_b,
               sf_a, sf_b, sg_a, sg_b):
    wid = lax.axis_index("s") * NC + lax.axis_index("c")
    nb = jnp.where(wid < EXTRA, BASE_NB + 1, BASE_NB)
    start = wid * BASE_NB + jnp.minimum(wid, EXTRA)
    row_start = start * BLK

    pltpu.sync_copy(pid_hbm.at[pl.ds(row_start, BASE_ROWS)],
                    idx_all.at[pl.ds(0, BASE_ROWS)])

    @pl.when(wid < EXTRA)
    def _():
        pltpu.sync_copy(pid_hbm.at[pl.ds(row_start + BASE_ROWS, BLK)],
                        idx_all.at[pl.ds(BASE_ROWS, BLK)])

    lane = lax.iota(jnp.int32, LANES)
    dnums = lax.GatherDimensionNumbers(
        offset_dims=(), collapsed_slice_dims=(0,), start_index_map=(0,))

    def allsum(v):
        for k in (8, 4, 2, 1):
            p = lax.gather(v, (lane ^ k)[:, None], dnums, (1,),
                           mode=lax.GatherScatterMode.PROMISE_IN_BOUNDS)
            v = v + p
        return v

    bufs = ((fb_a, rb_a, sf_a, sg_a),
            (fb_b, rb_b, sf_b, sg_b))

    def issue(b, p):
        fb, rb, sf, sg = bufs[p]
        pltpu.async_copy(feats_hbm.at[pl.ds(row_start + b * BLK, BLK)], fb, sf)
        pltpu.async_copy(mem_hbm.at[idx_all.at[pl.ds(b * BLK, BLK)]], rb, sg)

    def wait(p):
        fb, rb, sf, sg = bufs[p]
        pltpu.make_async_copy(feats_hbm.at[pl.ds(0, BLK)], fb, sf).wait()
        pltpu.make_async_copy(mem_hbm.at[idx_all.at[pl.ds(0, BLK)]],
                              rb, sg).wait()

    def one_instance(b, fb, rb, i, classes):
        zero = jnp.zeros((LANES,), jnp.float32)
        d_acc = zero
        f_acc = zero
        g_acc = zero
        half = D // 2
        for c in range(NCHUNK // 2):
            f_lo = fb[i, pl.ds(c * LANES, LANES)]
            f_hi = fb[i, pl.ds(half + c * LANES, LANES)]
            w = rb[i, pl.ds(c * LANES, LANES)]
            g_lo = lax.bitcast_convert_type(
                lax.shift_left(w, 16), jnp.float32)
            g_hi = lax.bitcast_convert_type(
                w & jnp.int32(-65536), jnp.float32)
            d_acc = d_acc + f_lo * g_lo + f_hi * g_hi
            f_acc = f_acc + f_lo * f_lo + f_hi * f_hi
            g_acc = g_acc + g_lo * g_lo + g_hi * g_hi
        dot = allsum(d_acc)
        fsq = allsum(f_acc)
        gsq = allsum(g_acc)
        t = fsq * gsq
        d2 = dot * dot
        pos = dot > 0.0
        is0 = pos & (d2 >= EPS_MODERATE_SQ * t)
        is1 = pos & (d2 >= EPS_PLAIN_SQ * t)
        cls = jnp.where(is0, 0, jnp.where(is1, 1, 2)).astype(jnp.int32)
        lane_in_grp = lax.rem(i, LANES)
        classes = jnp.where(lane == lane_in_grp, cls, classes)

        @pl.when((lane_in_grp == LANES - 1) | (i == BLK - 1))
        def _(i=i):
            grp = lax.div(i, LANES)
            out_all[pl.ds(b * BLK + grp * LANES, LANES)] = classes

        return classes

    def compute_block(b, p):
        fb, rb, _, _ = bufs[p]

        def inst_body(i2, classes):
            classes = one_instance(b, fb, rb, 2 * i2, classes)
            classes = one_instance(b, fb, rb, 2 * i2 + 1, classes)
            return classes

        lax.fori_loop(0, BLK // 2, inst_body, jnp.full((LANES,), 2, jnp.int32))

    issue(jnp.int32(0), 0)

    def pair_body(k, carry):
        b0 = 2 * k
        b1 = 2 * k + 1
        wait(0)

        @pl.when(b1 < nb)
        def _():
            issue(b1, 1)

        compute_block(b0, 0)

        @pl.when(b1 < nb)
        def _():
            wait(1)

            @pl.when(b1 + 1 < nb)
            def _():
                issue(b1 + 1, 0)

            compute_block(b1, 1)

        return carry

    lax.fori_loop(0, (nb + 1) // 2, pair_body, jnp.int32(0))

    pltpu.sync_copy(out_all.at[pl.ds(0, BASE_ROWS)],
                    out_hbm.at[pl.ds(row_start, BASE_ROWS)])

    @pl.when(wid < EXTRA)
    def _():
        pltpu.sync_copy(out_all.at[pl.ds(BASE_ROWS, BLK)],
                        out_hbm.at[pl.ds(row_start + BASE_ROWS, BLK)])


@jax.jit
def _tbgm(instance_feats, memory, pid2idx):
    mesh = plsc.VectorSubcoreMesh(core_axis_name="c", subcore_axis_name="s")
    fn = functools.partial(
        pl.kernel,
        out_type=jax.ShapeDtypeStruct((N,), jnp.int32),
        mesh=mesh,
        scratch_types=[
            pltpu.VMEM((MAX_ROWS,), jnp.int32),
            pltpu.VMEM((MAX_ROWS + NGRP * LANES,), jnp.int32),
            pltpu.VMEM((BLK, D), jnp.float32),
            pltpu.VMEM((BLK, D), jnp.float32),
            pltpu.VMEM((BLK, D // 2), jnp.int32),
            pltpu.VMEM((BLK, D // 2), jnp.int32),
            pltpu.SemaphoreType.DMA,
            pltpu.SemaphoreType.DMA,
            pltpu.SemaphoreType.DMA,
            pltpu.SemaphoreType.DMA,
        ],
    )(_tbgm_body)
    return fn(instance_feats, memory, pid2idx)


def kernel(instance_feats, memory, pid2idx):
    m_lo = memory[:, :D // 2].astype(jnp.bfloat16)
    m_hi = memory[:, D // 2:].astype(jnp.bfloat16)
    packed = jax.lax.bitcast_convert_type(
        jnp.stack([m_lo, m_hi], axis=-1), jnp.int32)
    return _tbgm(instance_feats, packed, pid2idx.astype(jnp.int32))


# speedup vs baseline: 1.2447x; 1.2447x over previous
import functools

import jax
import jax.numpy as jnp
from jax import lax
from jax.experimental import pallas as pl
from jax.experimental.pallas import tpu as pltpu
from jax.experimental.pallas import tpu_sc as plsc

N = 50000
C = 10000
D = 768
LANES = 16
BLK = 40
NUM_BLOCKS = N // BLK
NCHUNK = D // LANES
NGRP = (BLK + LANES - 1) // LANES
EPS_PLAIN_SQ = 0.4 * 0.4
EPS_MODERATE_SQ = 0.6 * 0.6

_info = plsc.get_sparse_core_info()
NC = _info.num_cores
NS = _info.num_subcores
NW = NC * NS
BASE_NB = NUM_BLOCKS // NW
EXTRA = NUM_BLOCKS % NW
MAX_NB = BASE_NB + 1
BASE_ROWS = BASE_NB * BLK
MAX_ROWS = MAX_NB * BLK


def _tbgm_body(feats_hbm, mem_hbm, pid_hbm, out_hbm,
               idx_all, out_all, fb_a, fb_b, rb_a, rb_b,
               sf_a, sf_b, sg_a, sg_b):
    wid = lax.axis_index("s") * NC + lax.axis_index("c")
    nb = jnp.where(wid < EXTRA, BASE_NB + 1, BASE_NB)
    start = wid * BASE_NB + jnp.minimum(wid, EXTRA)
    row_start = start * BLK

    pltpu.sync_copy(pid_hbm.at[pl.ds(row_start, BASE_ROWS)],
                    idx_all.at[pl.ds(0, BASE_ROWS)])

    @pl.when(wid < EXTRA)
    def _():
        pltpu.sync_copy(pid_hbm.at[pl.ds(row_start + BASE_ROWS, BLK)],
                        idx_all.at[pl.ds(BASE_ROWS, BLK)])

    lane = lax.iota(jnp.int32, LANES)
    dnums = lax.GatherDimensionNumbers(
        offset_dims=(), collapsed_slice_dims=(0,), start_index_map=(0,))

    def allsum(v):
        for k in (8, 4, 2, 1):
            p = lax.gather(v, (lane ^ k)[:, None], dnums, (1,),
                           mode=lax.GatherScatterMode.PROMISE_IN_BOUNDS)
            v = v + p
        return v

    bufs = ((fb_a, rb_a, sf_a, sg_a),
            (fb_b, rb_b, sf_b, sg_b))

    def issue(b, p):
        fb, rb, sf, sg = bufs[p]
        pltpu.async_copy(mem_hbm.at[idx_all.at[pl.ds(b * BLK, BLK)]], rb, sg)
        pltpu.async_copy(feats_hbm.at[pl.ds(row_start + b * BLK, BLK)], fb, sf)

    def wait(p):
        fb, rb, sf, sg = bufs[p]
        pltpu.make_async_copy(feats_hbm.at[pl.ds(0, BLK)], fb, sf).wait()
        pltpu.make_async_copy(mem_hbm.at[idx_all.at[pl.ds(0, BLK)]],
                              rb, sg).wait()

    def one_instance(b, fb, rb, i, classes):
        zero = jnp.zeros((LANES,), jnp.float32)
        d_acc = zero
        f_acc = zero
        g_acc = zero
        for c in range(NCHUNK):
            f = fb[i, pl.ds(c * LANES, LANES)]
            g = rb[i, pl.ds(c * LANES, LANES)]
            d_acc = d_acc + f * g
            f_acc = f_acc + f * f
            g_acc = g_acc + g * g
        dot = allsum(d_acc)
        fsq = allsum(f_acc)
        gsq = allsum(g_acc)
        t = fsq * gsq
        d2 = dot * dot
        pos = dot > 0.0
        is0 = pos & (d2 >= EPS_MODERATE_SQ * t)
        is1 = pos & (d2 >= EPS_PLAIN_SQ * t)
        cls = jnp.where(is0, 0, jnp.where(is1, 1, 2)).astype(jnp.int32)
        lane_in_grp = lax.rem(i, LANES)
        classes = jnp.where(lane == lane_in_grp, cls, classes)

        @pl.when((lane_in_grp == LANES - 1) | (i == BLK - 1))
        def _(i=i):
            grp = lax.div(i, LANES)
            out_all[pl.ds(b * BLK + grp * LANES, LANES)] = classes

        return classes

    def compute_block(b, p):
        fb, rb, _, _ = bufs[p]

        def inst_body(i2, classes):
            classes = one_instance(b, fb, rb, 2 * i2, classes)
            classes = one_instance(b, fb, rb, 2 * i2 + 1, classes)
            return classes

        lax.fori_loop(0, BLK // 2, inst_body, jnp.full((LANES,), 2, jnp.int32))

    issue(jnp.int32(0), 0)

    def pair_body(k, carry):
        b0 = 2 * k
        b1 = 2 * k + 1
        wait(0)

        @pl.when(b1 < nb)
        def _():
            issue(b1, 1)

        compute_block(b0, 0)

        @pl.when(b1 < nb)
        def _():
            wait(1)

            @pl.when(b1 + 1 < nb)
            def _():
                issue(b1 + 1, 0)

            compute_block(b1, 1)

        return carry

    lax.fori_loop(0, (nb + 1) // 2, pair_body, jnp.int32(0))

    pltpu.sync_copy(out_all.at[pl.ds(0, BASE_ROWS)],
                    out_hbm.at[pl.ds(row_start, BASE_ROWS)])

    @pl.when(wid < EXTRA)
    def _():
        pltpu.sync_copy(out_all.at[pl.ds(BASE_ROWS, BLK)],
                        out_hbm.at[pl.ds(row_start + BASE_ROWS, BLK)])


@jax.jit
def _tbgm(instance_feats, memory, pid2idx):
    mesh = plsc.VectorSubcoreMesh(core_axis_name="c", subcore_axis_name="s")
    fn = functools.partial(
        pl.kernel,
        out_type=jax.ShapeDtypeStruct((N,), jnp.int32),
        mesh=mesh,
        scratch_types=[
            pltpu.VMEM((MAX_ROWS,), jnp.int32),
            pltpu.VMEM((MAX_ROWS + NGRP * LANES,), jnp.int32),
            pltpu.VMEM((BLK, D), jnp.float32),
            pltpu.VMEM((BLK, D), jnp.float32),
            pltpu.VMEM((BLK, D), jnp.float32),
            pltpu.VMEM((BLK, D), jnp.float32),
            pltpu.SemaphoreType.DMA,
            pltpu.SemaphoreType.DMA,
            pltpu.SemaphoreType.DMA,
            pltpu.SemaphoreType.DMA,
        ],
    )(_tbgm_body)
    return fn(instance_feats, memory, pid2idx)


def kernel(instance_feats, memory, pid2idx):
    return _tbgm(instance_feats, memory, pid2idx.astype(jnp.int32))
